# baseline (speedup 1.0000x reference)
import functools

import jax
import jax.numpy as jnp
from jax import lax
from jax.experimental import pallas as pl
from jax.experimental.pallas import tpu as pltpu
from jax.experimental.pallas import tpu_sc as plsc

N_NODES = 50000
N_ENT = 40000
E = 800000
B = 8192

NC = 2
NS = 16
NW = NC * NS
CH = 128
E_PAD = ((E + NW * CH * 8 - 1) // (NW * CH * 8)) * (NW * CH * 8)
N_PAD = ((N_NODES + NS * 8 - 1) // (NS * 8)) * (NS * 8)
IDS_PAD = NW * CH * 8


def _segsum(ego, src2, dst2, w2, width):
    cpt = E_PAD // CH // NW
    GC = 8
    NG = cpt // GC
    NB = 4
    rpt = N_PAD // NS
    ZR = 136
    nz = rpt // ZR

    mesh = plsc.VectorSubcoreMesh(core_axis_name="c", subcore_axis_name="s")

    def body(ego_hbm, src_hbm, dst_hbm, w_hbm, out_hbm,
             src_v, dst_v, w_v, rows, zbuf, acc,
             zsem, isem_s, isem_d, isem_w,
             gsem0, gsem1, gsem2, gsem3, ssem0, ssem1, ssem2, ssem3):
        cid = lax.axis_index("c")
        sid = lax.axis_index("s")
        gsem = [gsem0, gsem1, gsem2, gsem3]
        ssem = [ssem0, ssem1, ssem2, ssem3]

        zero = jnp.zeros((16,), jnp.float32)

        def zrow(i, carry):
            for c in range(width // 16):
                zbuf[i, pl.ds(c * 16, 16)] = zero
            return carry
        lax.fori_loop(0, ZR, zrow, 0)

        def zcopy(j, carry):
            pltpu.async_copy(zbuf, acc.at[pl.ds(sid * rpt + j * ZR, ZR)], zsem)
            return carry
        lax.fori_loop(0, nz, zcopy, 0)

        wid = sid * NC + cid
        t0 = wid * cpt
        pltpu.async_copy(src_hbm.at[pl.ds(t0, GC)], src_v.at[0], isem_s)
        pltpu.async_copy(dst_hbm.at[pl.ds(t0, GC)], dst_v.at[0], isem_d)
        pltpu.async_copy(w_hbm.at[pl.ds(t0, GC)], w_v.at[0], isem_w)

        def zdrain(j, carry):
            pltpu.make_async_copy(
                zbuf, acc.at[pl.ds(sid * rpt, ZR)], zsem).wait()
            return carry
        lax.fori_loop(0, nz, zdrain, 0)
        plsc.subcore_barrier()

        pltpu.make_async_copy(src_hbm.at[pl.ds(t0, GC)],
                              src_v.at[0], isem_s).wait()
        pltpu.make_async_copy(dst_hbm.at[pl.ds(t0, GC)],
                              dst_v.at[0], isem_d).wait()
        pltpu.make_async_copy(w_hbm.at[pl.ds(t0, GC)],
                              w_v.at[0], isem_w).wait()

        pltpu.async_copy(ego_hbm.at[src_v.at[0, 0]], rows.at[0], gsem[0])
        pltpu.async_copy(ego_hbm.at[src_v.at[0, 1]], rows.at[1], gsem[1])

        def chunk(t, carry):
            gg = lax.shift_right_logical(t, 3)
            m = lax.bitwise_and(t, GC - 1)
            b = lax.bitwise_and(t, NB - 1)
            pb = lax.bitwise_and(gg, 1)
            not_last_group = gg < NG - 1

            @pl.when(jnp.logical_and(m == 3, not_last_group))
            def _prefetch():
                noff = t0 + (gg + 1) * GC
                pltpu.async_copy(src_hbm.at[pl.ds(noff, GC)],
                                 src_v.at[1 - pb], isem_s)
                pltpu.async_copy(dst_hbm.at[pl.ds(noff, GC)],
                                 dst_v.at[1 - pb], isem_d)
                pltpu.async_copy(w_hbm.at[pl.ds(noff, GC)],
                                 w_v.at[1 - pb], isem_w)

            @pl.when(jnp.logical_and(m == 5, not_last_group))
            def _iwait():
                pltpu.make_async_copy(src_hbm.at[pl.ds(t0, GC)],
                                      src_v.at[0], isem_s).wait()
                pltpu.make_async_copy(dst_hbm.at[pl.ds(t0, GC)],
                                      dst_v.at[0], isem_d).wait()
                pltpu.make_async_copy(w_hbm.at[pl.ds(t0, GC)],
                                      w_v.at[0], isem_w).wait()

            for i in range(NB):
                @pl.when(b == i)
                def _gwait(_i=i):
                    pltpu.make_async_copy(ego_hbm.at[pl.ds(0, CH)],
                                          rows.at[_i], gsem[_i]).wait()

            buf = rows.at[b]

            def mul(g, c2):
                wvec = w_v[pb, m, pl.ds(g * 16, 16)]
                base = g * 16
                for l in range(16):
                    wv = jnp.full((16,), wvec[l], jnp.float32)
                    for c in range(width // 16):
                        x = buf[base + l, pl.ds(c * 16, 16)]
                        buf[base + l, pl.ds(c * 16, 16)] = x * wv
                return c2
            lax.fori_loop(0, CH // 16, mul, 0)

            for i in range(NB):
                @pl.when(b == i)
                def _sissue(_i=i):
                    pltpu.async_copy(rows.at[_i], acc.at[dst_v.at[pb, m]],
                                     ssem[_i], add=True)

            t2 = t + 2
            b2 = lax.bitwise_and(t2, NB - 1)
            gg2 = lax.shift_right_logical(t2, 3)
            pb2 = lax.bitwise_and(gg2, 1)
            m2 = lax.bitwise_and(t2, GC - 1)
            for i in range(NB):
                @pl.when(jnp.logical_and(b2 == i, t >= 2))
                def _swait(_i=i):
                    pltpu.make_async_copy(rows.at[_i], acc.at[pl.ds(0, CH)],
                                          ssem[_i]).wait()
            for i in range(NB):
                @pl.when(jnp.logical_and(b2 == i, t2 < cpt))
                def _gnext(_i=i):
                    pltpu.async_copy(ego_hbm.at[src_v.at[pb2, m2]],
                                     rows.at[_i], gsem[_i])
            return carry
        lax.fori_loop(0, cpt, chunk, 0)

        pltpu.make_async_copy(rows.at[(cpt - 2) % NB],
                              acc.at[pl.ds(0, CH)],
                              ssem[(cpt - 2) % NB]).wait()
        pltpu.make_async_copy(rows.at[(cpt - 1) % NB],
                              acc.at[pl.ds(0, CH)],
                              ssem[(cpt - 1) % NB]).wait()

        plsc.subcore_barrier()
        pltpu.sync_copy(acc.at[pl.ds(sid * rpt, rpt)],
                        out_hbm.at[cid, pl.ds(sid * rpt, rpt)])

    return pl.kernel(
        body,
        out_type=jax.ShapeDtypeStruct((NC, N_PAD, width), jnp.float32),
        mesh=mesh,
        compiler_params=pltpu.CompilerParams(use_tc_tiling_on_sc=False),
        scratch_types=[
            pltpu.VMEM((2, GC, CH), jnp.int32),
            pltpu.VMEM((2, GC, CH), jnp.int32),
            pltpu.VMEM((2, GC, CH), jnp.float32),
            pltpu.VMEM((NB, CH, width), jnp.float32),
            pltpu.VMEM((ZR, width), jnp.float32),
            pltpu.VMEM_SHARED((N_PAD, width), jnp.float32),
        ] + [pltpu.SemaphoreType.DMA] * 12,
    )(ego, src2, dst2, w2)


def _combine(ego, side_parts, W1, b1, W2, b2):
    Di = ego.shape[1]
    Do = W1.shape[0]
    R = 2000
    G = N_NODES // R
    nparts = len(side_parts)

    def body(*refs):
        ego_ref = refs[0]
        side_refs = refs[1:1 + nparts]
        w1_ref, b1_ref, w2_ref, b2_ref, out_e, out_n = refs[1 + nparts:]
        e = ego_ref[...]
        side = jnp.concatenate([sr[0] + sr[1] for sr in side_refs], axis=1)
        s_in = e + side
        m_in = e * side
        dn = (((1,), (1,)), ((), ()))
        h1 = lax.dot_general(s_in, w1_ref[...], dn,
                             preferred_element_type=jnp.float32) + b1_ref[0:1, :]
        h2 = lax.dot_general(m_in, w2_ref[...], dn,
                             preferred_element_type=jnp.float32) + b2_ref[0:1, :]
        h1 = jnp.where(h1 >= 0, h1, 0.01 * h1)
        h2 = jnp.where(h2 >= 0, h2, 0.01 * h2)
        eo = h1 + h2
        out_e[...] = eo
        nrm = jnp.sqrt(jnp.sum(eo * eo, axis=1, keepdims=True))
        out_n[...] = eo / jnp.maximum(nrm, 1e-12)

    in_specs = [pl.BlockSpec((R, Di), lambda i: (i, 0))]
    for p in side_parts:
        Wp = p.shape[2]
        in_specs.append(pl.BlockSpec((2, R, Wp), lambda i: (0, i, 0)))
    in_specs += [
        pl.BlockSpec((Do, Di), lambda i: (0, 0)),
        pl.BlockSpec((8, Do), lambda i: (0, 0)),
        pl.BlockSpec((Do, Di), lambda i: (0, 0)),
        pl.BlockSpec((8, Do), lambda i: (0, 0)),
    ]
    out_specs = [pl.BlockSpec((R, Do), lambda i: (i, 0)),
                 pl.BlockSpec((R, Do), lambda i: (i, 0))]
    return pl.pallas_call(
        body,
        grid=(G,),
        in_specs=in_specs,
        out_specs=out_specs,
        out_shape=[jax.ShapeDtypeStruct((N_NODES, Do), jnp.float32)] * 2,
    )(ego, *side_parts, W1, jnp.broadcast_to(b1, (8, Do)),
      W2, jnp.broadcast_to(b2, (8, Do)))


def _gather4(t0a, t1a, t2a, t3a, ids2):
    total = ids2.shape[0] * CH
    per_tile = total // NW
    nch = per_tile // CH
    NBS = 4
    widths = [t0a.shape[1], t1a.shape[1], t2a.shape[1], t3a.shape[1]]

    mesh = plsc.VectorSubcoreMesh(core_axis_name="c", subcore_axis_name="s")

    def body(tbl0, tbl1, tbl2, tbl3, ids_hbm, out_p, idx_v,
             r0, r1, r2, r3,
             gsem0, gsem1, gsem2, gsem3, osem0, osem1, osem2, osem3):
        cid = lax.axis_index("c")
        sid = lax.axis_index("s")
        tbls = [tbl0, tbl1, tbl2, tbl3]
        rs = [r0, r1, r2, r3]
        gsem = [gsem0, gsem1, gsem2, gsem3]
        osem = [osem0, osem1, osem2, osem3]
        wid = sid * NC + cid
        pltpu.sync_copy(ids_hbm.at[pl.ds(wid * nch, nch)], idx_v)

        def issue_g(cc):
            b = cc % NBS
            for k in range(4):
                pltpu.async_copy(tbls[k].at[idx_v.at[cc]], rs[k].at[b],
                                 gsem[b])

        def wait_g(cc):
            b = cc % NBS
            for k in range(4):
                pltpu.make_async_copy(tbls[k].at[pl.ds(0, CH)], rs[k].at[b],
                                      gsem[b]).wait()

        cols = [0, widths[0], widths[0] + widths[1],
                widths[0] + widths[1] + widths[2]]

        def issue_o(cc):
            b = cc % NBS
            base = wid * per_tile + cc * CH
            for k in range(4):
                pltpu.async_copy(
                    rs[k].at[b],
                    out_p.at[pl.ds(base, CH), pl.ds(cols[k], widths[k])],
                    osem[b])

        def wait_o(cc):
            b = cc % NBS
            for k in range(4):
                pltpu.make_async_copy(
                    rs[k].at[b],
                    out_p.at[pl.ds(0, CH), pl.ds(cols[k], widths[k])],
                    osem[b]).wait()

        issue_g(0)
        issue_g(1)
        for cc in range(nch):
            wait_g(cc)
            issue_o(cc)
            if cc >= 2:
                wait_o(cc - 2)
            if cc + 2 < nch:
                issue_g(cc + 2)
        wait_o(nch - 2)
        wait_o(nch - 1)

    return pl.kernel(
        body,
        out_type=jax.ShapeDtypeStruct((total, 128), jnp.float32),
        mesh=mesh,
        compiler_params=pltpu.CompilerParams(use_tc_tiling_on_sc=False),
        scratch_types=[
            pltpu.VMEM((nch, CH), jnp.int32),
            pltpu.VMEM((NBS, CH, widths[0]), jnp.float32),
            pltpu.VMEM((NBS, CH, widths[1]), jnp.float32),
            pltpu.VMEM((NBS, CH, widths[2]), jnp.float32),
            pltpu.VMEM((NBS, CH, widths[3]), jnp.float32),
        ] + [pltpu.SemaphoreType.DMA] * 8,
    )(t0a, t1a, t2a, t3a, ids2)


def _loss_packed(rows):
    R = 2048
    G = B // R

    def body(ue_ref, pe_ref, ne_ref, out_ref):
        i = pl.program_id(0)
        ue = ue_ref[...]
        pe = pe_ref[...]
        ne = ne_ref[...]
        pos = jnp.sum(ue * pe, axis=1)
        neg = jnp.sum(ue * ne, axis=1)
        l2 = jnp.sum(ue * ue) + jnp.sum(pe * pe) + jnp.sum(ne * ne)
        x = pos - neg
        softplus_negx = jnp.maximum(-x, 0.0) + jnp.log1p(jnp.exp(-jnp.abs(x)))
        part = jnp.sum(softplus_negx) / B + 1e-5 * l2 / (2.0 * B)

        @pl.when(i == 0)
        def _init():
            out_ref[0, 0] = part

        @pl.when(i > 0)
        def _acc():
            out_ref[0, 0] = out_ref[0, 0] + part

    in_specs = [
        pl.BlockSpec((R, 128), lambda i: (i, 0)),
        pl.BlockSpec((R, 128), lambda i: (i + G, 0)),
        pl.BlockSpec((R, 128), lambda i: (i + 2 * G, 0)),
    ]
    return pl.pallas_call(
        body,
        grid=(G,),
        in_specs=in_specs,
        out_specs=pl.BlockSpec(memory_space=pltpu.SMEM),
        out_shape=jax.ShapeDtypeStruct((1, 1), jnp.float32),
    )(rows, rows, rows)


def kernel(user_ids, item_pos_ids, item_neg_ids, edge_index, table, edge_weight,
           W1_0, b1_0, W2_0, b2_0, W1_1, b1_1, W2_1, b2_1, W1_2, b1_2, W2_2, b2_2):
    src = edge_index[0].astype(jnp.int32)
    dst = edge_index[1].astype(jnp.int32)
    pad = E_PAD - E
    src2 = jnp.concatenate([src, jnp.zeros((pad,), jnp.int32)]).reshape(-1, CH)
    dst2 = jnp.concatenate([dst, jnp.zeros((pad,), jnp.int32)]).reshape(-1, CH)
    w2 = jnp.concatenate([edge_weight.astype(jnp.float32),
                          jnp.zeros((pad,), jnp.float32)]).reshape(-1, CH)

    slo = _segsum(table[:, :32], src2, dst2, w2, 32)
    shi = _segsum(table[:, 32:], src2, dst2, w2, 32)
    ego1, n1 = _combine(table, [slo, shi], W1_0, b1_0, W2_0, b2_0)

    s1 = _segsum(ego1, src2, dst2, w2, 32)
    ego2, n2 = _combine(ego1, [s1], W1_1, b1_1, W2_1, b2_1)

    s2 = _segsum(ego2, src2, dst2, w2, 16)
    ego3, n3 = _combine(ego2, [s2], W1_2, b1_2, W2_2, b2_2)

    n3p = jnp.concatenate([n3, jnp.zeros((N_NODES, 8), jnp.float32)], axis=1)
    cat_ids = jnp.concatenate([user_ids, item_pos_ids, item_neg_ids]
                              ).astype(jnp.int32).reshape(-1, CH)
    rows = _gather4(table, n1, n2, n3p, cat_ids)
    return _loss_packed(rows)[0, 0]


# speedup vs baseline: 1.0328x; 1.0328x over previous
import functools

import jax
import jax.numpy as jnp
from jax import lax
from jax.experimental import pallas as pl
from jax.experimental.pallas import tpu as pltpu
from jax.experimental.pallas import tpu_sc as plsc

N_NODES = 50000
N_ENT = 40000
E = 800000
B = 8192

NC = 2
NS = 16
NW = NC * NS
CH = 128
E_PAD = ((E + NW * CH * 8 - 1) // (NW * CH * 8)) * (NW * CH * 8)
N_PAD = ((N_NODES + NS * 8 - 1) // (NS * 8)) * (NS * 8)
IDS_PAD = NW * CH * 8


def _segsum(ego, src2, dst2, w2, width):
    cpt = E_PAD // CH // NW
    GC = 8
    NG = cpt // GC
    NB = 4
    rpt = N_PAD // NS
    ZR = 136
    nz = rpt // ZR

    mesh = plsc.VectorSubcoreMesh(core_axis_name="c", subcore_axis_name="s")

    def body(ego_hbm, src_hbm, dst_hbm, w_hbm, out_hbm,
             src_v, dst_v, w_v, rows, zbuf, acc,
             zsem, isem_s, isem_d, isem_w,
             gsem0, gsem1, gsem2, gsem3, ssem0, ssem1, ssem2, ssem3):
        cid = lax.axis_index("c")
        sid = lax.axis_index("s")
        gsem = [gsem0, gsem1, gsem2, gsem3]
        ssem = [ssem0, ssem1, ssem2, ssem3]

        zero = jnp.zeros((16,), jnp.float32)

        def zrow(i, carry):
            for c in range(width // 16):
                zbuf[i, pl.ds(c * 16, 16)] = zero
            return carry
        lax.fori_loop(0, ZR, zrow, 0)

        def zcopy(j, carry):
            pltpu.async_copy(zbuf, acc.at[pl.ds(sid * rpt + j * ZR, ZR)], zsem)
            return carry
        lax.fori_loop(0, nz, zcopy, 0)

        wid = sid * NC + cid
        t0 = wid * cpt
        pltpu.async_copy(src_hbm.at[pl.ds(t0, GC)], src_v.at[0], isem_s)
        pltpu.async_copy(dst_hbm.at[pl.ds(t0, GC)], dst_v.at[0], isem_d)
        pltpu.async_copy(w_hbm.at[pl.ds(t0, GC)], w_v.at[0], isem_w)

        def zdrain(j, carry):
            pltpu.make_async_copy(
                zbuf, acc.at[pl.ds(sid * rpt, ZR)], zsem).wait()
            return carry
        lax.fori_loop(0, nz, zdrain, 0)
        plsc.subcore_barrier()

        pltpu.make_async_copy(src_hbm.at[pl.ds(t0, GC)],
                              src_v.at[0], isem_s).wait()
        pltpu.make_async_copy(dst_hbm.at[pl.ds(t0, GC)],
                              dst_v.at[0], isem_d).wait()
        pltpu.make_async_copy(w_hbm.at[pl.ds(t0, GC)],
                              w_v.at[0], isem_w).wait()

        pltpu.async_copy(ego_hbm.at[src_v.at[0, 0]], rows.at[0], gsem[0])
        pltpu.async_copy(ego_hbm.at[src_v.at[0, 1]], rows.at[1], gsem[1])

        def chunk(t, carry):
            gg = lax.shift_right_logical(t, 3)
            m = lax.bitwise_and(t, GC - 1)
            b = lax.bitwise_and(t, NB - 1)
            pb = lax.bitwise_and(gg, 1)
            not_last_group = gg < NG - 1

            @pl.when(jnp.logical_and(m == 3, not_last_group))
            def _prefetch():
                noff = t0 + (gg + 1) * GC
                pltpu.async_copy(src_hbm.at[pl.ds(noff, GC)],
                                 src_v.at[1 - pb], isem_s)
                pltpu.async_copy(dst_hbm.at[pl.ds(noff, GC)],
                                 dst_v.at[1 - pb], isem_d)
                pltpu.async_copy(w_hbm.at[pl.ds(noff, GC)],
                                 w_v.at[1 - pb], isem_w)

            @pl.when(jnp.logical_and(m == 5, not_last_group))
            def _iwait():
                pltpu.make_async_copy(src_hbm.at[pl.ds(t0, GC)],
                                      src_v.at[0], isem_s).wait()
                pltpu.make_async_copy(dst_hbm.at[pl.ds(t0, GC)],
                                      dst_v.at[0], isem_d).wait()
                pltpu.make_async_copy(w_hbm.at[pl.ds(t0, GC)],
                                      w_v.at[0], isem_w).wait()

            for i in range(NB):
                @pl.when(b == i)
                def _gwait(_i=i):
                    pltpu.make_async_copy(ego_hbm.at[pl.ds(0, CH)],
                                          rows.at[_i], gsem[_i]).wait()

            buf = rows.at[b]

            def mul(g, c2):
                wvec = w_v[pb, m, pl.ds(g * 16, 16)]
                base = g * 16
                for l in range(16):
                    wv = jnp.full((16,), wvec[l], jnp.float32)
                    for c in range(width // 16):
                        x = buf[base + l, pl.ds(c * 16, 16)]
                        buf[base + l, pl.ds(c * 16, 16)] = x * wv
                return c2
            lax.fori_loop(0, CH // 16, mul, 0)

            for i in range(NB):
                @pl.when(b == i)
                def _sissue(_i=i):
                    pltpu.async_copy(rows.at[_i], acc.at[dst_v.at[pb, m]],
                                     ssem[_i], add=True)

            t2 = t + 2
            b2 = lax.bitwise_and(t2, NB - 1)
            gg2 = lax.shift_right_logical(t2, 3)
            pb2 = lax.bitwise_and(gg2, 1)
            m2 = lax.bitwise_and(t2, GC - 1)
            for i in range(NB):
                @pl.when(jnp.logical_and(b2 == i, t >= 2))
                def _swait(_i=i):
                    pltpu.make_async_copy(rows.at[_i], acc.at[pl.ds(0, CH)],
                                          ssem[_i]).wait()
            for i in range(NB):
                @pl.when(jnp.logical_and(b2 == i, t2 < cpt))
                def _gnext(_i=i):
                    pltpu.async_copy(ego_hbm.at[src_v.at[pb2, m2]],
                                     rows.at[_i], gsem[_i])
            return carry
        lax.fori_loop(0, cpt, chunk, 0)

        pltpu.make_async_copy(rows.at[(cpt - 2) % NB],
                              acc.at[pl.ds(0, CH)],
                              ssem[(cpt - 2) % NB]).wait()
        pltpu.make_async_copy(rows.at[(cpt - 1) % NB],
                              acc.at[pl.ds(0, CH)],
                              ssem[(cpt - 1) % NB]).wait()

        plsc.subcore_barrier()
        pltpu.sync_copy(acc.at[pl.ds(sid * rpt, rpt)],
                        out_hbm.at[cid, pl.ds(sid * rpt, rpt)])

    return pl.kernel(
        body,
        out_type=jax.ShapeDtypeStruct((NC, N_PAD, width), jnp.float32),
        mesh=mesh,
        compiler_params=pltpu.CompilerParams(use_tc_tiling_on_sc=False),
        scratch_types=[
            pltpu.VMEM((2, GC, CH), jnp.int32),
            pltpu.VMEM((2, GC, CH), jnp.int32),
            pltpu.VMEM((2, GC, CH), jnp.float32),
            pltpu.VMEM((NB, CH, width), jnp.float32),
            pltpu.VMEM((ZR, width), jnp.float32),
            pltpu.VMEM_SHARED((N_PAD, width), jnp.float32),
        ] + [pltpu.SemaphoreType.DMA] * 12,
    )(ego, src2, dst2, w2)


def _combine(ego, side_parts, W1, b1, W2, b2):
    Di = ego.shape[1]
    Do = W1.shape[0]
    R = 2000
    G = N_NODES // R
    nparts = len(side_parts)

    def body(*refs):
        ego_ref = refs[0]
        side_refs = refs[1:1 + nparts]
        w1_ref, b1_ref, w2_ref, b2_ref, out_e, out_n = refs[1 + nparts:]
        e = ego_ref[...]
        side = jnp.concatenate([sr[0] + sr[1] for sr in side_refs], axis=1)
        s_in = e + side
        m_in = e * side
        dn = (((1,), (1,)), ((), ()))
        h1 = lax.dot_general(s_in, w1_ref[...], dn,
                             preferred_element_type=jnp.float32) + b1_ref[0:1, :]
        h2 = lax.dot_general(m_in, w2_ref[...], dn,
                             preferred_element_type=jnp.float32) + b2_ref[0:1, :]
        h1 = jnp.where(h1 >= 0, h1, 0.01 * h1)
        h2 = jnp.where(h2 >= 0, h2, 0.01 * h2)
        eo = h1 + h2
        out_e[...] = eo
        nrm = jnp.sqrt(jnp.sum(eo * eo, axis=1, keepdims=True))
        out_n[...] = eo / jnp.maximum(nrm, 1e-12)

    in_specs = [pl.BlockSpec((R, Di), lambda i: (i, 0))]
    for p in side_parts:
        Wp = p.shape[2]
        in_specs.append(pl.BlockSpec((2, R, Wp), lambda i: (0, i, 0)))
    in_specs += [
        pl.BlockSpec((Do, Di), lambda i: (0, 0)),
        pl.BlockSpec((8, Do), lambda i: (0, 0)),
        pl.BlockSpec((Do, Di), lambda i: (0, 0)),
        pl.BlockSpec((8, Do), lambda i: (0, 0)),
    ]
    out_specs = [pl.BlockSpec((R, Do), lambda i: (i, 0)),
                 pl.BlockSpec((R, Do), lambda i: (i, 0))]
    return pl.pallas_call(
        body,
        grid=(G,),
        in_specs=in_specs,
        out_specs=out_specs,
        out_shape=[jax.ShapeDtypeStruct((N_NODES, Do), jnp.float32)] * 2,
    )(ego, *side_parts, W1, jnp.broadcast_to(b1, (8, Do)),
      W2, jnp.broadcast_to(b2, (8, Do)))


def _gather4(t0a, t1a, t2a, t3a, ids2):
    total = ids2.shape[0] * CH
    per_tile = total // NW
    nch = per_tile // CH
    NBS = 4
    widths = [t0a.shape[1], t1a.shape[1], t2a.shape[1], t3a.shape[1]]

    mesh = plsc.VectorSubcoreMesh(core_axis_name="c", subcore_axis_name="s")

    def body(tbl0, tbl1, tbl2, tbl3, ids_hbm, o0, o1, o2, o3, idx_v,
             r0, r1, r2, r3,
             gsem0, gsem1, gsem2, gsem3, osem0, osem1, osem2, osem3):
        cid = lax.axis_index("c")
        sid = lax.axis_index("s")
        tbls = [tbl0, tbl1, tbl2, tbl3]
        outs = [o0, o1, o2, o3]
        rs = [r0, r1, r2, r3]
        gsem = [gsem0, gsem1, gsem2, gsem3]
        osem = [osem0, osem1, osem2, osem3]
        wid = sid * NC + cid
        pltpu.sync_copy(ids_hbm.at[pl.ds(wid * nch, nch)], idx_v)

        def issue_g(cc):
            b = cc % NBS
            for k in range(4):
                pltpu.async_copy(tbls[k].at[idx_v.at[cc]], rs[k].at[b],
                                 gsem[b])

        def wait_g(cc):
            b = cc % NBS
            for k in range(4):
                pltpu.make_async_copy(tbls[k].at[pl.ds(0, CH)], rs[k].at[b],
                                      gsem[b]).wait()

        def issue_o(cc):
            b = cc % NBS
            base = wid * per_tile + cc * CH
            for k in range(4):
                pltpu.async_copy(rs[k].at[b], outs[k].at[pl.ds(base, CH)],
                                 osem[b])

        def wait_o(cc):
            b = cc % NBS
            for k in range(4):
                pltpu.make_async_copy(rs[k].at[b],
                                      outs[k].at[pl.ds(0, CH)], osem[b]).wait()

        issue_g(0)
        issue_g(1)
        for cc in range(nch):
            wait_g(cc)
            issue_o(cc)
            if cc >= 2:
                wait_o(cc - 2)
            if cc + 2 < nch:
                issue_g(cc + 2)
        wait_o(nch - 2)
        wait_o(nch - 1)

    return pl.kernel(
        body,
        out_type=[jax.ShapeDtypeStruct((total, w), jnp.float32)
                  for w in widths],
        mesh=mesh,
        compiler_params=pltpu.CompilerParams(use_tc_tiling_on_sc=False),
        scratch_types=[
            pltpu.VMEM((nch, CH), jnp.int32),
            pltpu.VMEM((NBS, CH, widths[0]), jnp.float32),
            pltpu.VMEM((NBS, CH, widths[1]), jnp.float32),
            pltpu.VMEM((NBS, CH, widths[2]), jnp.float32),
            pltpu.VMEM((NBS, CH, widths[3]), jnp.float32),
        ] + [pltpu.SemaphoreType.DMA] * 8,
    )(t0a, t1a, t2a, t3a, ids2)


def _loss4(g0, g1, g2, g3):
    R = 1024
    G = B // R

    def body(*refs):
        i = pl.program_id(0)
        out_ref = refs[-1]
        pos = jnp.zeros((R,), jnp.float32)
        neg = jnp.zeros((R,), jnp.float32)
        l2 = jnp.float32(0.0)
        for k in range(4):
            ue = refs[3 * k][...]
            pe = refs[3 * k + 1][...]
            ne = refs[3 * k + 2][...]
            pos = pos + jnp.sum(ue * pe, axis=1)
            neg = neg + jnp.sum(ue * ne, axis=1)
            l2 = l2 + (jnp.sum(ue * ue) + jnp.sum(pe * pe)
                       + jnp.sum(ne * ne))
        x = pos - neg
        softplus_negx = jnp.maximum(-x, 0.0) + jnp.log1p(jnp.exp(-jnp.abs(x)))
        part = jnp.sum(softplus_negx) / B + 1e-5 * l2 / (2.0 * B)

        @pl.when(i == 0)
        def _init():
            out_ref[0, 0] = part

        @pl.when(i > 0)
        def _acc():
            out_ref[0, 0] = out_ref[0, 0] + part

    tables = [g0, g1, g2, g3]
    in_specs = []
    args = []
    for k, t in enumerate(tables):
        w = t.shape[1]
        in_specs += [
            pl.BlockSpec((R, w), lambda i: (i, 0)),
            pl.BlockSpec((R, w), lambda i: (i + G, 0)),
            pl.BlockSpec((R, w), lambda i: (i + 2 * G, 0)),
        ]
        args += [t, t, t]
    return pl.pallas_call(
        body,
        grid=(G,),
        in_specs=in_specs,
        out_specs=pl.BlockSpec(memory_space=pltpu.SMEM),
        out_shape=jax.ShapeDtypeStruct((1, 1), jnp.float32),
    )(*args)


def kernel(user_ids, item_pos_ids, item_neg_ids, edge_index, table, edge_weight,
           W1_0, b1_0, W2_0, b2_0, W1_1, b1_1, W2_1, b2_1, W1_2, b1_2, W2_2, b2_2):
    src = edge_index[0].astype(jnp.int32)
    dst = edge_index[1].astype(jnp.int32)
    pad = E_PAD - E
    src2 = jnp.concatenate([src, jnp.zeros((pad,), jnp.int32)]).reshape(-1, CH)
    dst2 = jnp.concatenate([dst, jnp.zeros((pad,), jnp.int32)]).reshape(-1, CH)
    w2 = jnp.concatenate([edge_weight.astype(jnp.float32),
                          jnp.zeros((pad,), jnp.float32)]).reshape(-1, CH)

    slo = _segsum(table[:, :32], src2, dst2, w2, 32)
    shi = _segsum(table[:, 32:], src2, dst2, w2, 32)
    ego1, n1 = _combine(table, [slo, shi], W1_0, b1_0, W2_0, b2_0)

    s1 = _segsum(ego1, src2, dst2, w2, 32)
    ego2, n2 = _combine(ego1, [s1], W1_1, b1_1, W2_1, b2_1)

    s2 = _segsum(ego2, src2, dst2, w2, 16)
    ego3, n3 = _combine(ego2, [s2], W1_2, b1_2, W2_2, b2_2)

    n3p = jnp.concatenate([n3, jnp.zeros((N_NODES, 8), jnp.float32)], axis=1)
    cat_ids = jnp.concatenate([user_ids, item_pos_ids, item_neg_ids]
                              ).astype(jnp.int32).reshape(-1, CH)
    g0, g1, g2, g3 = _gather4(table, n1, n2, n3p, cat_ids)
    return _loss4(g0, g1, g2, g3)[0, 0]


# speedup vs baseline: 1.0331x; 1.0004x over previous
import functools

import jax
import jax.numpy as jnp
from jax import lax
from jax.experimental import pallas as pl
from jax.experimental.pallas import tpu as pltpu
from jax.experimental.pallas import tpu_sc as plsc

N_NODES = 50000
N_ENT = 40000
E = 800000
B = 8192

NC = 2
NS = 16
NW = NC * NS
CH = 128
E_PAD = ((E + NW * CH * 8 - 1) // (NW * CH * 8)) * (NW * CH * 8)
N_PAD = ((N_NODES + NS * 8 - 1) // (NS * 8)) * (NS * 8)
IDS_PAD = NW * CH * 8


def _segsum(ego, src2, dst2, w2, width):
    cpt = E_PAD // CH // NW
    GC = 8
    NG = cpt // GC
    NB = 4
    rpt = N_PAD // NS
    ZR = 136
    nz = rpt // ZR

    mesh = plsc.VectorSubcoreMesh(core_axis_name="c", subcore_axis_name="s")

    def body(ego_hbm, src_hbm, dst_hbm, w_hbm, out_hbm,
             src_v, dst_v, w_v, rows, zbuf, acc,
             zsem, isem_s, isem_d, isem_w,
             gsem0, gsem1, gsem2, gsem3, ssem0, ssem1, ssem2, ssem3):
        cid = lax.axis_index("c")
        sid = lax.axis_index("s")
        gsem = [gsem0, gsem1, gsem2, gsem3]
        ssem = [ssem0, ssem1, ssem2, ssem3]

        zero = jnp.zeros((16,), jnp.float32)

        def zrow(i, carry):
            for c in range(width // 16):
                zbuf[i, pl.ds(c * 16, 16)] = zero
            return carry
        lax.fori_loop(0, ZR, zrow, 0)

        def zcopy(j, carry):
            pltpu.async_copy(zbuf, acc.at[pl.ds(sid * rpt + j * ZR, ZR)], zsem)
            return carry
        lax.fori_loop(0, nz, zcopy, 0)

        wid = sid * NC + cid
        t0 = wid * cpt
        pltpu.async_copy(src_hbm.at[pl.ds(t0, GC)], src_v.at[0], isem_s)
        pltpu.async_copy(dst_hbm.at[pl.ds(t0, GC)], dst_v.at[0], isem_d)
        pltpu.async_copy(w_hbm.at[pl.ds(t0, GC)], w_v.at[0], isem_w)

        def zdrain(j, carry):
            pltpu.make_async_copy(
                zbuf, acc.at[pl.ds(sid * rpt, ZR)], zsem).wait()
            return carry
        lax.fori_loop(0, nz, zdrain, 0)
        plsc.subcore_barrier()

        pltpu.make_async_copy(src_hbm.at[pl.ds(t0, GC)],
                              src_v.at[0], isem_s).wait()
        pltpu.make_async_copy(dst_hbm.at[pl.ds(t0, GC)],
                              dst_v.at[0], isem_d).wait()
        pltpu.make_async_copy(w_hbm.at[pl.ds(t0, GC)],
                              w_v.at[0], isem_w).wait()

        pltpu.async_copy(ego_hbm.at[src_v.at[0, 0]], rows.at[0], gsem[0])
        pltpu.async_copy(ego_hbm.at[src_v.at[0, 1]], rows.at[1], gsem[1])

        def chunk(t, carry):
            gg = lax.shift_right_logical(t, 3)
            m = lax.bitwise_and(t, GC - 1)
            b = lax.bitwise_and(t, NB - 1)
            pb = lax.bitwise_and(gg, 1)
            not_last_group = gg < NG - 1

            @pl.when(jnp.logical_and(m == 3, not_last_group))
            def _prefetch():
                noff = t0 + (gg + 1) * GC
                pltpu.async_copy(src_hbm.at[pl.ds(noff, GC)],
                                 src_v.at[1 - pb], isem_s)
                pltpu.async_copy(dst_hbm.at[pl.ds(noff, GC)],
                                 dst_v.at[1 - pb], isem_d)
                pltpu.async_copy(w_hbm.at[pl.ds(noff, GC)],
                                 w_v.at[1 - pb], isem_w)

            @pl.when(jnp.logical_and(m == 5, not_last_group))
            def _iwait():
                pltpu.make_async_copy(src_hbm.at[pl.ds(t0, GC)],
                                      src_v.at[0], isem_s).wait()
                pltpu.make_async_copy(dst_hbm.at[pl.ds(t0, GC)],
                                      dst_v.at[0], isem_d).wait()
                pltpu.make_async_copy(w_hbm.at[pl.ds(t0, GC)],
                                      w_v.at[0], isem_w).wait()

            for i in range(NB):
                @pl.when(b == i)
                def _gwait(_i=i):
                    pltpu.make_async_copy(ego_hbm.at[pl.ds(0, CH)],
                                          rows.at[_i], gsem[_i]).wait()

            buf = rows.at[b]

            def mul(g, c2):
                wvec = w_v[pb, m, pl.ds(g * 16, 16)]
                base = g * 16
                for l in range(16):
                    wv = jnp.full((16,), wvec[l], jnp.float32)
                    for c in range(width // 16):
                        x = buf[base + l, pl.ds(c * 16, 16)]
                        buf[base + l, pl.ds(c * 16, 16)] = x * wv
                return c2
            lax.fori_loop(0, CH // 16, mul, 0)

            for i in range(NB):
                @pl.when(b == i)
                def _sissue(_i=i):
                    pltpu.async_copy(rows.at[_i], acc.at[dst_v.at[pb, m]],
                                     ssem[_i], add=True)

            t2 = t + 2
            b2 = lax.bitwise_and(t2, NB - 1)
            gg2 = lax.shift_right_logical(t2, 3)
            pb2 = lax.bitwise_and(gg2, 1)
            m2 = lax.bitwise_and(t2, GC - 1)
            for i in range(NB):
                @pl.when(jnp.logical_and(b2 == i, t >= 2))
                def _swait(_i=i):
                    pltpu.make_async_copy(rows.at[_i], acc.at[pl.ds(0, CH)],
                                          ssem[_i]).wait()
            for i in range(NB):
                @pl.when(jnp.logical_and(b2 == i, t2 < cpt))
                def _gnext(_i=i):
                    pltpu.async_copy(ego_hbm.at[src_v.at[pb2, m2]],
                                     rows.at[_i], gsem[_i])
            return carry
        lax.fori_loop(0, cpt, chunk, 0)

        pltpu.make_async_copy(rows.at[(cpt - 2) % NB],
                              acc.at[pl.ds(0, CH)],
                              ssem[(cpt - 2) % NB]).wait()
        pltpu.make_async_copy(rows.at[(cpt - 1) % NB],
                              acc.at[pl.ds(0, CH)],
                              ssem[(cpt - 1) % NB]).wait()

        plsc.subcore_barrier()
        pltpu.sync_copy(acc.at[pl.ds(sid * rpt, rpt)],
                        out_hbm.at[cid, pl.ds(sid * rpt, rpt)])

    return pl.kernel(
        body,
        out_type=jax.ShapeDtypeStruct((NC, N_PAD, width), jnp.float32),
        mesh=mesh,
        compiler_params=pltpu.CompilerParams(use_tc_tiling_on_sc=False),
        scratch_types=[
            pltpu.VMEM((2, GC, CH), jnp.int32),
            pltpu.VMEM((2, GC, CH), jnp.int32),
            pltpu.VMEM((2, GC, CH), jnp.float32),
            pltpu.VMEM((NB, CH, width), jnp.float32),
            pltpu.VMEM((ZR, width), jnp.float32),
            pltpu.VMEM_SHARED((N_PAD, width), jnp.float32),
        ] + [pltpu.SemaphoreType.DMA] * 12,
    )(ego, src2, dst2, w2)


def _combine(ego, side_parts, W1, b1, W2, b2):
    Di = ego.shape[1]
    Do = W1.shape[0]
    R = 2000
    G = N_NODES // R
    nparts = len(side_parts)

    def body(*refs):
        ego_ref = refs[0]
        side_refs = refs[1:1 + nparts]
        w1_ref, b1_ref, w2_ref, b2_ref, out_e, out_n = refs[1 + nparts:]
        e = ego_ref[...]
        side = jnp.concatenate([sr[0] + sr[1] for sr in side_refs], axis=1)
        s_in = e + side
        m_in = e * side
        dn = (((1,), (1,)), ((), ()))
        h1 = lax.dot_general(s_in, w1_ref[...], dn,
                             preferred_element_type=jnp.float32) + b1_ref[0:1, :]
        h2 = lax.dot_general(m_in, w2_ref[...], dn,
                             preferred_element_type=jnp.float32) + b2_ref[0:1, :]
        h1 = jnp.where(h1 >= 0, h1, 0.01 * h1)
        h2 = jnp.where(h2 >= 0, h2, 0.01 * h2)
        eo = h1 + h2
        out_e[...] = eo
        nrm = jnp.sqrt(jnp.sum(eo * eo, axis=1, keepdims=True))
        out_n[...] = eo / jnp.maximum(nrm, 1e-12)

    in_specs = [pl.BlockSpec((R, Di), lambda i: (i, 0))]
    for p in side_parts:
        Wp = p.shape[2]
        in_specs.append(pl.BlockSpec((2, R, Wp), lambda i: (0, i, 0)))
    in_specs += [
        pl.BlockSpec((Do, Di), lambda i: (0, 0)),
        pl.BlockSpec((8, Do), lambda i: (0, 0)),
        pl.BlockSpec((Do, Di), lambda i: (0, 0)),
        pl.BlockSpec((8, Do), lambda i: (0, 0)),
    ]
    out_specs = [pl.BlockSpec((R, Do), lambda i: (i, 0)),
                 pl.BlockSpec((R, Do), lambda i: (i, 0))]
    return pl.pallas_call(
        body,
        grid=(G,),
        in_specs=in_specs,
        out_specs=out_specs,
        out_shape=[jax.ShapeDtypeStruct((N_NODES, Do), jnp.float32)] * 2,
    )(ego, *side_parts, W1, jnp.broadcast_to(b1, (8, Do)),
      W2, jnp.broadcast_to(b2, (8, Do)))


def _gather4(t0a, t1a, t2a, t3a, ids2):
    total = ids2.shape[0] * CH
    per_tile = total // NW
    nch = per_tile // CH
    NBS = 4
    widths = [t0a.shape[1], t1a.shape[1], t2a.shape[1], t3a.shape[1]]

    mesh = plsc.VectorSubcoreMesh(core_axis_name="c", subcore_axis_name="s")

    def body(tbl0, tbl1, tbl2, tbl3, ids_hbm, o0, o1, o2, o3, idx_v,
             r0, r1, r2, r3,
             gsem0, gsem1, gsem2, gsem3, osem0, osem1, osem2, osem3):
        cid = lax.axis_index("c")
        sid = lax.axis_index("s")
        tbls = [tbl0, tbl1, tbl2, tbl3]
        outs = [o0, o1, o2, o3]
        rs = [r0, r1, r2, r3]
        gsem = [gsem0, gsem1, gsem2, gsem3]
        osem = [osem0, osem1, osem2, osem3]
        wid = sid * NC + cid
        pltpu.sync_copy(ids_hbm.at[pl.ds(wid * nch, nch)], idx_v)

        def issue_g(cc):
            b = cc % NBS
            for k in range(4):
                pltpu.async_copy(tbls[k].at[idx_v.at[cc]], rs[k].at[b],
                                 gsem[b])

        def wait_g(cc):
            b = cc % NBS
            for k in range(4):
                pltpu.make_async_copy(tbls[k].at[pl.ds(0, CH)], rs[k].at[b],
                                      gsem[b]).wait()

        def issue_o(cc):
            b = cc % NBS
            base = wid * per_tile + cc * CH
            for k in range(4):
                pltpu.async_copy(rs[k].at[b], outs[k].at[pl.ds(base, CH)],
                                 osem[b])

        def wait_o(cc):
            b = cc % NBS
            for k in range(4):
                pltpu.make_async_copy(rs[k].at[b],
                                      outs[k].at[pl.ds(0, CH)], osem[b]).wait()

        issue_g(0)
        issue_g(1)
        for cc in range(nch):
            wait_g(cc)
            issue_o(cc)
            if cc >= 2:
                wait_o(cc - 2)
            if cc + 2 < nch:
                issue_g(cc + 2)
        wait_o(nch - 2)
        wait_o(nch - 1)

    return pl.kernel(
        body,
        out_type=[jax.ShapeDtypeStruct((total, w), jnp.float32)
                  for w in widths],
        mesh=mesh,
        compiler_params=pltpu.CompilerParams(use_tc_tiling_on_sc=False),
        scratch_types=[
            pltpu.VMEM((nch, CH), jnp.int32),
            pltpu.VMEM((NBS, CH, widths[0]), jnp.float32),
            pltpu.VMEM((NBS, CH, widths[1]), jnp.float32),
            pltpu.VMEM((NBS, CH, widths[2]), jnp.float32),
            pltpu.VMEM((NBS, CH, widths[3]), jnp.float32),
        ] + [pltpu.SemaphoreType.DMA] * 8,
    )(t0a, t1a, t2a, t3a, ids2)


def _loss4(g0, g1, g2, g3):
    R = 2048
    G = B // R

    def body(*refs):
        i = pl.program_id(0)
        out_ref = refs[-1]
        pos = jnp.zeros((R,), jnp.float32)
        neg = jnp.zeros((R,), jnp.float32)
        l2 = jnp.float32(0.0)
        for k in range(4):
            ue = refs[3 * k][...]
            pe = refs[3 * k + 1][...]
            ne = refs[3 * k + 2][...]
            pos = pos + jnp.sum(ue * pe, axis=1)
            neg = neg + jnp.sum(ue * ne, axis=1)
            l2 = l2 + (jnp.sum(ue * ue) + jnp.sum(pe * pe)
                       + jnp.sum(ne * ne))
        x = pos - neg
        softplus_negx = jnp.maximum(-x, 0.0) + jnp.log1p(jnp.exp(-jnp.abs(x)))
        part = jnp.sum(softplus_negx) / B + 1e-5 * l2 / (2.0 * B)

        @pl.when(i == 0)
        def _init():
            out_ref[0, 0] = part

        @pl.when(i > 0)
        def _acc():
            out_ref[0, 0] = out_ref[0, 0] + part

    tables = [g0, g1, g2, g3]
    in_specs = []
    args = []
    for k, t in enumerate(tables):
        w = t.shape[1]
        in_specs += [
            pl.BlockSpec((R, w), lambda i: (i, 0)),
            pl.BlockSpec((R, w), lambda i: (i + G, 0)),
            pl.BlockSpec((R, w), lambda i: (i + 2 * G, 0)),
        ]
        args += [t, t, t]
    return pl.pallas_call(
        body,
        grid=(G,),
        in_specs=in_specs,
        out_specs=pl.BlockSpec(memory_space=pltpu.SMEM),
        out_shape=jax.ShapeDtypeStruct((1, 1), jnp.float32),
    )(*args)


def kernel(user_ids, item_pos_ids, item_neg_ids, edge_index, table, edge_weight,
           W1_0, b1_0, W2_0, b2_0, W1_1, b1_1, W2_1, b2_1, W1_2, b1_2, W2_2, b2_2):
    src = edge_index[0].astype(jnp.int32)
    dst = edge_index[1].astype(jnp.int32)
    pad = E_PAD - E
    src2 = jnp.concatenate([src, jnp.zeros((pad,), jnp.int32)]).reshape(-1, CH)
    dst2 = jnp.concatenate([dst, jnp.zeros((pad,), jnp.int32)]).reshape(-1, CH)
    w2 = jnp.concatenate([edge_weight.astype(jnp.float32),
                          jnp.zeros((pad,), jnp.float32)]).reshape(-1, CH)

    slo = _segsum(table[:, :32], src2, dst2, w2, 32)
    shi = _segsum(table[:, 32:], src2, dst2, w2, 32)
    ego1, n1 = _combine(table, [slo, shi], W1_0, b1_0, W2_0, b2_0)

    s1 = _segsum(ego1, src2, dst2, w2, 32)
    ego2, n2 = _combine(ego1, [s1], W1_1, b1_1, W2_1, b2_1)

    s2 = _segsum(ego2, src2, dst2, w2, 16)
    ego3, n3 = _combine(ego2, [s2], W1_2, b1_2, W2_2, b2_2)

    n3p = jnp.concatenate([n3, jnp.zeros((N_NODES, 8), jnp.float32)], axis=1)
    cat_ids = jnp.concatenate([user_ids, item_pos_ids, item_neg_ids]
                              ).astype(jnp.int32).reshape(-1, CH)
    g0, g1, g2, g3 = _gather4(table, n1, n2, n3p, cat_ids)
    return _loss4(g0, g1, g2, g3)[0, 0]


# speedup vs baseline: 1.0333x; 1.0001x over previous
import jax
import jax.numpy as jnp
from jax import lax
from jax.experimental import pallas as pl
from jax.experimental.pallas import tpu as pltpu
from jax.experimental.pallas import tpu_sc as plsc

N_NODES = 50000
E = 800000
B = 8192

NC = 2
NS = 16
NW = NC * NS
CH = 128
E_PAD = ((E + NW * CH * 8 - 1) // (NW * CH * 8)) * (NW * CH * 8)
N_PAD = ((N_NODES + NS * 8 - 1) // (NS * 8)) * (NS * 8)


def _segsum(ego, src2, dst2, w2, width):
    cpt = E_PAD // CH // NW
    GC = 8
    NG = cpt // GC
    NB = 4
    rpt = N_PAD // NS
    ZR = 136
    nz = rpt // ZR

    mesh = plsc.VectorSubcoreMesh(core_axis_name="c", subcore_axis_name="s")

    def body(ego_hbm, src_hbm, dst_hbm, w_hbm, out_hbm,
             src_v, dst_v, w_v, rows, zbuf, acc,
             zsem, isem_s, isem_d, isem_w,
             gsem0, gsem1, gsem2, gsem3, ssem0, ssem1, ssem2, ssem3):
        cid = lax.axis_index("c")
        sid = lax.axis_index("s")
        gsem = [gsem0, gsem1, gsem2, gsem3]
        ssem = [ssem0, ssem1, ssem2, ssem3]

        zero = jnp.zeros((16,), jnp.float32)

        def zrow(i, carry):
            for c in range(width // 16):
                zbuf[i, pl.ds(c * 16, 16)] = zero
            return carry
        lax.fori_loop(0, ZR, zrow, 0)

        def zcopy(j, carry):
            pltpu.async_copy(zbuf, acc.at[pl.ds(sid * rpt + j * ZR, ZR)], zsem)
            return carry
        lax.fori_loop(0, nz, zcopy, 0)

        wid = sid * NC + cid
        t0 = wid * cpt
        pltpu.async_copy(src_hbm.at[pl.ds(t0, GC)], src_v.at[0], isem_s)
        pltpu.async_copy(dst_hbm.at[pl.ds(t0, GC)], dst_v.at[0], isem_d)
        pltpu.async_copy(w_hbm.at[pl.ds(t0, GC)], w_v.at[0], isem_w)

        def zdrain(j, carry):
            pltpu.make_async_copy(
                zbuf, acc.at[pl.ds(sid * rpt, ZR)], zsem).wait()
            return carry
        lax.fori_loop(0, nz, zdrain, 0)
        plsc.subcore_barrier()

        pltpu.make_async_copy(src_hbm.at[pl.ds(t0, GC)],
                              src_v.at[0], isem_s).wait()
        pltpu.make_async_copy(dst_hbm.at[pl.ds(t0, GC)],
                              dst_v.at[0], isem_d).wait()
        pltpu.make_async_copy(w_hbm.at[pl.ds(t0, GC)],
                              w_v.at[0], isem_w).wait()

        pltpu.async_copy(ego_hbm.at[src_v.at[0, 0]], rows.at[0], gsem[0])
        pltpu.async_copy(ego_hbm.at[src_v.at[0, 1]], rows.at[1], gsem[1])

        def chunk(t, carry):
            gg = lax.shift_right_logical(t, 3)
            m = lax.bitwise_and(t, GC - 1)
            b = lax.bitwise_and(t, NB - 1)
            pb = lax.bitwise_and(gg, 1)
            not_last_group = gg < NG - 1

            @pl.when(jnp.logical_and(m == 3, not_last_group))
            def _prefetch():
                noff = t0 + (gg + 1) * GC
                pltpu.async_copy(src_hbm.at[pl.ds(noff, GC)],
                                 src_v.at[1 - pb], isem_s)
                pltpu.async_copy(dst_hbm.at[pl.ds(noff, GC)],
                                 dst_v.at[1 - pb], isem_d)
                pltpu.async_copy(w_hbm.at[pl.ds(noff, GC)],
                                 w_v.at[1 - pb], isem_w)

            @pl.when(jnp.logical_and(m == 5, not_last_group))
            def _iwait():
                pltpu.make_async_copy(src_hbm.at[pl.ds(t0, GC)],
                                      src_v.at[0], isem_s).wait()
                pltpu.make_async_copy(dst_hbm.at[pl.ds(t0, GC)],
                                      dst_v.at[0], isem_d).wait()
                pltpu.make_async_copy(w_hbm.at[pl.ds(t0, GC)],
                                      w_v.at[0], isem_w).wait()

            for i in range(NB):
                @pl.when(b == i)
                def _gwait(_i=i):
                    pltpu.make_async_copy(ego_hbm.at[pl.ds(0, CH)],
                                          rows.at[_i], gsem[_i]).wait()

            buf = rows.at[b]

            def mul(g, c2):
                wvec = w_v[pb, m, pl.ds(g * 16, 16)]
                base = g * 16
                for l in range(16):
                    wv = jnp.full((16,), wvec[l], jnp.float32)
                    for c in range(width // 16):
                        x = buf[base + l, pl.ds(c * 16, 16)]
                        buf[base + l, pl.ds(c * 16, 16)] = x * wv
                return c2
            lax.fori_loop(0, CH // 16, mul, 0)

            for i in range(NB):
                @pl.when(b == i)
                def _sissue(_i=i):
                    pltpu.async_copy(rows.at[_i], acc.at[dst_v.at[pb, m]],
                                     ssem[_i], add=True)

            t2 = t + 2
            b2 = lax.bitwise_and(t2, NB - 1)
            gg2 = lax.shift_right_logical(t2, 3)
            pb2 = lax.bitwise_and(gg2, 1)
            m2 = lax.bitwise_and(t2, GC - 1)
            for i in range(NB):
                @pl.when(jnp.logical_and(b2 == i, t >= 2))
                def _swait(_i=i):
                    pltpu.make_async_copy(rows.at[_i], acc.at[pl.ds(0, CH)],
                                          ssem[_i]).wait()
            for i in range(NB):
                @pl.when(jnp.logical_and(b2 == i, t2 < cpt))
                def _gnext(_i=i):
                    pltpu.async_copy(ego_hbm.at[src_v.at[pb2, m2]],
                                     rows.at[_i], gsem[_i])
            return carry
        lax.fori_loop(0, cpt, chunk, 0)

        pltpu.make_async_copy(rows.at[(cpt - 2) % NB],
                              acc.at[pl.ds(0, CH)],
                              ssem[(cpt - 2) % NB]).wait()
        pltpu.make_async_copy(rows.at[(cpt - 1) % NB],
                              acc.at[pl.ds(0, CH)],
                              ssem[(cpt - 1) % NB]).wait()

        plsc.subcore_barrier()
        pltpu.sync_copy(acc.at[pl.ds(sid * rpt, rpt)],
                        out_hbm.at[cid, pl.ds(sid * rpt, rpt)])

    return pl.kernel(
        body,
        out_type=jax.ShapeDtypeStruct((NC, N_PAD, width), jnp.float32),
        mesh=mesh,
        compiler_params=pltpu.CompilerParams(use_tc_tiling_on_sc=False),
        scratch_types=[
            pltpu.VMEM((2, GC, CH), jnp.int32),
            pltpu.VMEM((2, GC, CH), jnp.int32),
            pltpu.VMEM((2, GC, CH), jnp.float32),
            pltpu.VMEM((NB, CH, width), jnp.float32),
            pltpu.VMEM((ZR, width), jnp.float32),
            pltpu.VMEM_SHARED((N_PAD, width), jnp.float32),
        ] + [pltpu.SemaphoreType.DMA] * 12,
    )(ego, src2, dst2, w2)


def _combine(ego, side_parts, W1, b1, W2, b2):
    Di = ego.shape[1]
    Do = W1.shape[0]
    R = 2000
    G = N_NODES // R
    nparts = len(side_parts)

    def body(*refs):
        ego_ref = refs[0]
        side_refs = refs[1:1 + nparts]
        w1_ref, b1_ref, w2_ref, b2_ref, out_e, out_n = refs[1 + nparts:]
        e = ego_ref[...]
        side = jnp.concatenate([sr[0] + sr[1] for sr in side_refs], axis=1)
        s_in = e + side
        m_in = e * side
        dn = (((1,), (1,)), ((), ()))
        h1 = lax.dot_general(s_in, w1_ref[...], dn,
                             preferred_element_type=jnp.float32) + b1_ref[0:1, :]
        h2 = lax.dot_general(m_in, w2_ref[...], dn,
                             preferred_element_type=jnp.float32) + b2_ref[0:1, :]
        h1 = jnp.where(h1 >= 0, h1, 0.01 * h1)
        h2 = jnp.where(h2 >= 0, h2, 0.01 * h2)
        eo = h1 + h2
        out_e[...] = eo
        nrm = jnp.sqrt(jnp.sum(eo * eo, axis=1, keepdims=True))
        out_n[...] = eo / jnp.maximum(nrm, 1e-12)

    in_specs = [pl.BlockSpec((R, Di), lambda i: (i, 0))]
    for p in side_parts:
        Wp = p.shape[2]
        in_specs.append(pl.BlockSpec((2, R, Wp), lambda i: (0, i, 0)))
    in_specs += [
        pl.BlockSpec((Do, Di), lambda i: (0, 0)),
        pl.BlockSpec((8, Do), lambda i: (0, 0)),
        pl.BlockSpec((Do, Di), lambda i: (0, 0)),
        pl.BlockSpec((8, Do), lambda i: (0, 0)),
    ]
    out_specs = [pl.BlockSpec((R, Do), lambda i: (i, 0)),
                 pl.BlockSpec((R, Do), lambda i: (i, 0))]
    return pl.pallas_call(
        body,
        grid=(G,),
        in_specs=in_specs,
        out_specs=out_specs,
        out_shape=[jax.ShapeDtypeStruct((N_NODES, Do), jnp.float32)] * 2,
    )(ego, *side_parts, W1, jnp.broadcast_to(b1, (8, Do)),
      W2, jnp.broadcast_to(b2, (8, Do)))


def _gather4(t0a, t1a, t2a, t3a, ids2):
    total = ids2.shape[0] * CH
    per_tile = total // NW
    nch = per_tile // CH
    NBS = 4
    widths = [t0a.shape[1], t1a.shape[1], t2a.shape[1], t3a.shape[1]]

    mesh = plsc.VectorSubcoreMesh(core_axis_name="c", subcore_axis_name="s")

    def body(tbl0, tbl1, tbl2, tbl3, ids_hbm, o0, o1, o2, o3, idx_v,
             r0, r1, r2, r3,
             gsem0, gsem1, gsem2, gsem3, osem0, osem1, osem2, osem3):
        cid = lax.axis_index("c")
        sid = lax.axis_index("s")
        tbls = [tbl0, tbl1, tbl2, tbl3]
        outs = [o0, o1, o2, o3]
        rs = [r0, r1, r2, r3]
        gsem = [gsem0, gsem1, gsem2, gsem3]
        osem = [osem0, osem1, osem2, osem3]
        wid = sid * NC + cid
        pltpu.sync_copy(ids_hbm.at[pl.ds(wid * nch, nch)], idx_v)

        def issue_g(cc):
            b = cc % NBS
            for k in range(4):
                pltpu.async_copy(tbls[k].at[idx_v.at[cc]], rs[k].at[b],
                                 gsem[b])

        def wait_g(cc):
            b = cc % NBS
            for k in range(4):
                pltpu.make_async_copy(tbls[k].at[pl.ds(0, CH)], rs[k].at[b],
                                      gsem[b]).wait()

        def issue_o(cc):
            b = cc % NBS
            base = wid * per_tile + cc * CH
            for k in range(4):
                pltpu.async_copy(rs[k].at[b], outs[k].at[pl.ds(base, CH)],
                                 osem[b])

        def wait_o(cc):
            b = cc % NBS
            for k in range(4):
                pltpu.make_async_copy(rs[k].at[b],
                                      outs[k].at[pl.ds(0, CH)], osem[b]).wait()

        issue_g(0)
        issue_g(1)
        for cc in range(nch):
            wait_g(cc)
            issue_o(cc)
            if cc >= 2:
                wait_o(cc - 2)
            if cc + 2 < nch:
                issue_g(cc + 2)
        wait_o(nch - 2)
        wait_o(nch - 1)

    return pl.kernel(
        body,
        out_type=[jax.ShapeDtypeStruct((total, w), jnp.float32)
                  for w in widths],
        mesh=mesh,
        compiler_params=pltpu.CompilerParams(use_tc_tiling_on_sc=False),
        scratch_types=[
            pltpu.VMEM((nch, CH), jnp.int32),
            pltpu.VMEM((NBS, CH, widths[0]), jnp.float32),
            pltpu.VMEM((NBS, CH, widths[1]), jnp.float32),
            pltpu.VMEM((NBS, CH, widths[2]), jnp.float32),
            pltpu.VMEM((NBS, CH, widths[3]), jnp.float32),
        ] + [pltpu.SemaphoreType.DMA] * 8,
    )(t0a, t1a, t2a, t3a, ids2)


def _loss4(g0, g1, g2, g3):
    R = 2048
    G = B // R

    def body(*refs):
        i = pl.program_id(0)
        out_ref = refs[-1]
        pos = jnp.zeros((R,), jnp.float32)
        neg = jnp.zeros((R,), jnp.float32)
        l2 = jnp.float32(0.0)
        for k in range(4):
            ue = refs[3 * k][...]
            pe = refs[3 * k + 1][...]
            ne = refs[3 * k + 2][...]
            pos = pos + jnp.sum(ue * pe, axis=1)
            neg = neg + jnp.sum(ue * ne, axis=1)
            l2 = l2 + (jnp.sum(ue * ue) + jnp.sum(pe * pe)
                       + jnp.sum(ne * ne))
        x = pos - neg
        softplus_negx = jnp.maximum(-x, 0.0) + jnp.log1p(jnp.exp(-jnp.abs(x)))
        part = jnp.sum(softplus_negx) / B + 1e-5 * l2 / (2.0 * B)

        @pl.when(i == 0)
        def _init():
            out_ref[0, 0] = part

        @pl.when(i > 0)
        def _acc():
            out_ref[0, 0] = out_ref[0, 0] + part

    tables = [g0, g1, g2, g3]
    in_specs = []
    args = []
    for k, t in enumerate(tables):
        w = t.shape[1]
        in_specs += [
            pl.BlockSpec((R, w), lambda i: (i, 0)),
            pl.BlockSpec((R, w), lambda i: (i + G, 0)),
            pl.BlockSpec((R, w), lambda i: (i + 2 * G, 0)),
        ]
        args += [t, t, t]
    return pl.pallas_call(
        body,
        grid=(G,),
        in_specs=in_specs,
        out_specs=pl.BlockSpec(memory_space=pltpu.SMEM),
        out_shape=jax.ShapeDtypeStruct((1, 1), jnp.float32),
    )(*args)


def kernel(user_ids, item_pos_ids, item_neg_ids, edge_index, table, edge_weight,
           W1_0, b1_0, W2_0, b2_0, W1_1, b1_1, W2_1, b2_1, W1_2, b1_2, W2_2, b2_2):
    src = edge_index[0].astype(jnp.int32)
    dst = edge_index[1].astype(jnp.int32)
    pad = E_PAD - E
    src2 = jnp.concatenate([src, jnp.zeros((pad,), jnp.int32)]).reshape(-1, CH)
    dst2 = jnp.concatenate([dst, jnp.zeros((pad,), jnp.int32)]).reshape(-1, CH)
    w2 = jnp.concatenate([edge_weight.astype(jnp.float32),
                          jnp.zeros((pad,), jnp.float32)]).reshape(-1, CH)

    slo = _segsum(table[:, :32], src2, dst2, w2, 32)
    shi = _segsum(table[:, 32:], src2, dst2, w2, 32)
    ego1, n1 = _combine(table, [slo, shi], W1_0, b1_0, W2_0, b2_0)

    s1 = _segsum(ego1, src2, dst2, w2, 32)
    ego2, n2 = _combine(ego1, [s1], W1_1, b1_1, W2_1, b2_1)

    s2 = _segsum(ego2, src2, dst2, w2, 16)
    ego3, n3 = _combine(ego2, [s2], W1_2, b1_2, W2_2, b2_2)

    n3p = jnp.concatenate([n3, jnp.zeros((N_NODES, 8), jnp.float32)], axis=1)
    cat_ids = jnp.concatenate([user_ids, item_pos_ids, item_neg_ids]
                              ).astype(jnp.int32).reshape(-1, CH)
    g0, g1, g2, g3 = _gather4(table, n1, n2, n3p, cat_ids)
    return _loss4(g0, g1, g2, g3)[0, 0]
